# Initial kernel scaffold; baseline (speedup 1.0000x reference)
import jax
import jax.numpy as jnp
from jax.experimental import pallas as pl


def kernel(t, xh, node_mask, edge_mask, gcl_e_W1, gcl_e_b1, gcl_e_W2, gcl_e_b2, gcl_n_W1, gcl_n_b1, gcl_n_W2, gcl_n_b2, eq_W1, eq_b1, eq_W2, eq_b2, eq_W3, emb_W, emb_b, out_W, out_b):
    raise NotImplementedError("write your pallas kernel here")


# speedup vs baseline: 17.6519x; 17.6519x over previous
import functools

import jax
import jax.numpy as jnp
from jax.experimental import pallas as pl

_BT = 8
_NORM = 100.0
_F32 = jnp.float32


def _silu(v):
    return v * jax.lax.logistic(v)


def _body(t_ref, xh_ref, nm_ref, em_ref,
          geW1h_ref, gew1c_ref, geb1_ref, geW2_ref, geb2_ref,
          gnW1h_ref, gnW1a_ref, gnb1_ref, gnW2_ref, gnb2_ref,
          eqW1h_ref, eqw1c_ref, eqb1_ref, eqW2_ref, eqb2_ref, eqw3_ref,
          embW_ref, embb_ref, outW_ref, outb_ref,
          out_ref, *, bt, nn, hid, n_layers, inv_sub):
    n = bt * nn
    E = bt * nn * nn
    dims = xh_ref.shape[-1]
    hd = dims - 3

    def row_exp(v):
        F = v.shape[1]
        return jnp.broadcast_to(v.reshape(n, 1, F), (n, nn, F)).reshape(E, F)

    def col_exp(v):
        F = v.shape[1]
        return jnp.broadcast_to(v.reshape(bt, 1, nn, F),
                                (bt, nn, nn, F)).reshape(E, F)

    def seg(e):
        return jnp.sum(e.reshape(n, nn, e.shape[1]), axis=1)

    def dot(a, b):
        return jnp.dot(a, b, preferred_element_type=_F32)

    nm = nm_ref[...].reshape(n, 1)
    em = em_ref[...]
    xh_f = xh_ref[...].reshape(n, dims) * nm
    x0 = xh_f[:, :3]
    h5 = xh_f[:, 3:]
    h_time = jnp.broadcast_to(t_ref[...].reshape(bt, 1, 1),
                              (bt, nn, 1)).reshape(n, 1)
    h = jnp.concatenate([h5, h_time], axis=1)
    h = dot(h, embW_ref[...]) + embb_ref[...]

    geW1h = geW1h_ref[...]
    gew1c = gew1c_ref[...]
    geb1 = geb1_ref[...]
    geW2 = geW2_ref[...]
    geb2 = geb2_ref[...]
    gnW1h = gnW1h_ref[...]
    gnW1a = gnW1a_ref[...]
    gnb1 = gnb1_ref[...]
    gnW2 = gnW2_ref[...]
    gnb2 = gnb2_ref[...]
    eqW1h = eqW1h_ref[...]
    eqw1c = eqw1c_ref[...]
    eqb1 = eqb1_ref[...]
    eqW2 = eqW2_ref[...]
    eqb2 = eqb2_ref[...]
    eqw3 = eqw3_ref[...]

    def pair_radial(x):
        xr = row_exp(x)
        xc = col_exp(x)
        diff = xr - xc
        radial = jnp.sum(diff * diff, axis=1, keepdims=True)
        return radial, diff

    dist0, _ = pair_radial(x0)
    x = x0
    for l in range(n_layers):
        dist_l, diff = pair_radial(x)
        coord_diff = diff / jnp.sqrt(dist_l + 1e-8)
        for s in range(inv_sub):
            g = l * inv_sub + s
            hrc = dot(h, geW1h[g])
            pre = (row_exp(hrc[:, :hid]) + col_exp(hrc[:, hid:])
                   + dist_l * gew1c[g, 0:1, :] + dist0 * gew1c[g, 1:2, :]
                   + geb1[g:g + 1, :])
            mij = _silu(dot(_silu(pre), geW2[g]) + geb2[g:g + 1, :]) * em
            agg = seg(mij) * (1.0 / _NORM)
            nin = dot(h, gnW1h[g]) + dot(agg, gnW1a[g]) + gnb1[g:g + 1, :]
            h = (h + dot(_silu(nin), gnW2[g]) + gnb2[g:g + 1, :]) * nm
        hrc = dot(h, eqW1h[l])
        pre = (row_exp(hrc[:, :hid]) + col_exp(hrc[:, hid:])
               + dist_l * eqw1c[l, 0:1, :] + dist0 * eqw1c[l, 1:2, :]
               + eqb1[l:l + 1, :])
        m = _silu(dot(_silu(pre), eqW2[l]) + eqb2[l:l + 1, :])
        sval = jnp.sum(m * eqw3[l:l + 1, :], axis=1, keepdims=True)
        trans = coord_diff * sval * em
        x = (x + seg(trans) * (1.0 / _NORM)) * nm
        h = h * nm

    hf = (dot(h, outW_ref[...]) + outb_ref[...]) * nm
    vel = (x - x0) * nm
    vel3 = vel.reshape(bt, nn, 3)
    nm3 = nm.reshape(bt, nn, 1)
    n_per = jnp.sum(nm3, axis=1, keepdims=True)
    vel3 = vel3 - (jnp.sum(vel3, axis=1, keepdims=True) / n_per) * nm3
    out_ref[...] = jnp.concatenate([vel3, hf.reshape(bt, nn, hd)], axis=2)


def kernel(t, xh, node_mask, edge_mask, gcl_e_W1, gcl_e_b1, gcl_e_W2,
           gcl_e_b2, gcl_n_W1, gcl_n_b1, gcl_n_W2, gcl_n_b2, eq_W1, eq_b1,
           eq_W2, eq_b2, eq_W3, emb_W, emb_b, out_W, out_b):
    bs, nn, dims = xh.shape
    hid = gcl_e_W2.shape[-1]
    hd = dims - 3
    n_layers = eq_W1.shape[0]
    inv_sub = gcl_e_W1.shape[0] // n_layers
    bt = _BT
    grid = bs // bt

    geW1h = jnp.concatenate([gcl_e_W1[:, :hid, :], gcl_e_W1[:, hid:2 * hid, :]],
                            axis=2)
    gew1c = gcl_e_W1[:, 2 * hid:, :]
    eqW1h = jnp.concatenate([eq_W1[:, :hid, :], eq_W1[:, hid:2 * hid, :]],
                            axis=2)
    eqw1c = eq_W1[:, 2 * hid:, :]
    gnW1h = gcl_n_W1[:, :hid, :]
    gnW1a = gcl_n_W1[:, hid:, :]
    eqw3 = eq_W3[:, :, 0]
    embb = emb_b.reshape(1, -1)
    outW = out_W[:, :hd]
    outb = out_b[:hd].reshape(1, -1)

    def wspec(a):
        nd = a.ndim
        return pl.BlockSpec(a.shape, lambda i, nd=nd: (0,) * nd)

    weights = (geW1h, gew1c, gcl_e_b1, gcl_e_W2, gcl_e_b2,
               gnW1h, gnW1a, gcl_n_b1, gcl_n_W2, gcl_n_b2,
               eqW1h, eqw1c, eq_b1, eq_W2, eq_b2, eqw3,
               emb_W, embb, outW, outb)

    body = functools.partial(_body, bt=bt, nn=nn, hid=hid,
                             n_layers=n_layers, inv_sub=inv_sub)
    out = pl.pallas_call(
        body,
        grid=(grid,),
        in_specs=[
            pl.BlockSpec((bt, 1), lambda i: (i, 0)),
            pl.BlockSpec((bt, nn, dims), lambda i: (i, 0, 0)),
            pl.BlockSpec((bt, nn, 1), lambda i: (i, 0, 0)),
            pl.BlockSpec((bt * nn * nn, 1), lambda i: (i, 0)),
        ] + [wspec(w) for w in weights],
        out_specs=pl.BlockSpec((bt, nn, dims), lambda i: (i, 0, 0)),
        out_shape=jax.ShapeDtypeStruct((bs, nn, dims), _F32),
    )(t, xh, node_mask, edge_mask, *weights)
    return out
